# baseline (speedup 1.0000x reference)
import jax
import jax.numpy as jnp
from jax import lax
from jax.experimental import pallas as pl
from jax.experimental.pallas import tpu as pltpu
from jax.experimental.pallas import tpu_sc as plsc

B, T, V, D, DU = 1024, 50, 100000, 128, 256
U = D + DU
TP1 = T + 1
NNZ_SEQ = 2
NNZ_SENT = 4
NW = 32
L = 16
CB = 32
NCHUNK = B // CB
NUNITS = TP1 * NCHUNK
UPW = NUNITS // NW
EPT = B // 16


def _sc_body(seqi3, seqd_t, sent_idx, sent_dense, lens_b, wseq, wsent,
             comb_t, masko,
             sidx_st, gsh, dsh, lensh, srow_st, mst,
             idx_pl, g0, g1, dch, sch, lch, o,
             srow_all,
             sem1, sg0, sg1):
    cid = lax.axis_index("c")
    sid = lax.axis_index("s")
    wid = sid * 2 + cid

    iota = jnp.arange(L, dtype=jnp.int32)
    zero_f = jnp.zeros((L,), jnp.float32)
    one_f = jnp.float32(1.0)
    zero_s = jnp.float32(0.0)

    b1 = sid * EPT
    pltpu.sync_copy(sent_idx.at[pl.ds(b1 * NNZ_SENT, EPT * NNZ_SENT)],
                    sidx_st)
    for h in range(2):
        hb = b1 + h * 32
        pltpu.async_copy(
            wsent.at[sidx_st.at[pl.ds(h * 128, 128)]], gsh, sem1).wait()
        pltpu.sync_copy(sent_dense.at[pl.ds(hb, 32)], dsh)
        pltpu.sync_copy(lens_b.at[pl.ds(hb, 32)], lensh)

        def ph1_body(j2, carry):
            for cc in range(D // L):
                srow_st[j2, pl.ds(cc * L, L)] = (
                    (gsh[NNZ_SENT * j2, pl.ds(cc * L, L)]
                     + gsh[NNZ_SENT * j2 + 1, pl.ds(cc * L, L)])
                    + (gsh[NNZ_SENT * j2 + 2, pl.ds(cc * L, L)]
                       + gsh[NNZ_SENT * j2 + 3, pl.ds(cc * L, L)]))
            for cc in range(DU // L):
                srow_st[j2, pl.ds(D + cc * L, L)] = dsh[j2, pl.ds(cc * L, L)]
            len_v = lensh[j2, pl.ds(0, L)]
            lp1 = jnp.full((L,), 1, jnp.int32) + len_v
            for kk in range(4):
                tvec = kk * L + iota
                m = jnp.where(tvec < lp1, one_f, zero_s)
                mst[pl.ds((h * 32 + j2) * TP1 + kk * L, L)] = m
            return carry

        lax.fori_loop(0, 32, ph1_body, 0)
        pltpu.sync_copy(srow_st, srow_all.at[pl.ds(hb, 32)])

    @pl.when(cid == 0)
    def _():
        pltpu.sync_copy(mst.at[pl.ds(0, EPT * TP1)],
                        masko.at[pl.ds(b1 * TP1, EPT * TP1)])

    plsc.subcore_barrier()

    def unit_body(u, carry):
        t = u // NCHUNK
        c = u - t * NCHUNK
        cq = c // 4
        r0 = (c - cq * 4) * 32

        @pl.when(t < T)
        def _():
            pltpu.sync_copy(seqi3.at[t], idx_pl)
            pltpu.async_copy(
                wseq.at[idx_pl.at[2 * cq, pl.ds(r0, 32)]], g0, sg0)
            pltpu.async_copy(
                wseq.at[idx_pl.at[2 * cq + 1, pl.ds(r0, 32)]], g1, sg1)
            pltpu.sync_copy(seqd_t.at[t, pl.ds(CB * c, CB)], dch)
            pltpu.make_async_copy(
                wseq.at[idx_pl.at[2 * cq, pl.ds(r0, 32)]], g0, sg0).wait()
            pltpu.make_async_copy(
                wseq.at[idx_pl.at[2 * cq + 1, pl.ds(r0, 32)]], g1,
                sg1).wait()

        pltpu.sync_copy(srow_all.at[pl.ds(CB * c, CB)], sch)
        pltpu.sync_copy(lens_b.at[pl.ds(CB * c, CB)], lch)
        tv = jnp.full((L,), t, jnp.int32)

        @plsc.parallel_loop(0, CB, unroll=2)
        def row_body(bl):
            len_v = lch[bl, pl.ds(0, L)]
            m_lt = tv < len_v
            m_eq = tv == len_v
            for cc in range(D // L):
                v = g0[bl, pl.ds(cc * L, L)] + g1[bl, pl.ds(cc * L, L)]
                o[bl, pl.ds(cc * L, L)] = jnp.where(
                    m_lt, v, jnp.where(m_eq, sch[bl, pl.ds(cc * L, L)],
                                       zero_f))
            for cc in range(DU // L):
                o[bl, pl.ds(D + cc * L, L)] = jnp.where(
                    m_lt, dch[bl, pl.ds(cc * L, L)],
                    jnp.where(m_eq, sch[bl, pl.ds(D + cc * L, L)], zero_f))

        pltpu.sync_copy(o, comb_t.at[t, pl.ds(CB * c, CB)])
        return carry

    lax.fori_loop(wid * UPW, (wid + 1) * UPW, unit_body, 0)


@jax.jit
def _run(seqi3, seqd_t, sent_idx, sent_dense, lens_b, wseq, wsent):
    mesh = plsc.VectorSubcoreMesh(core_axis_name="c", subcore_axis_name="s")
    return pl.kernel(
        _sc_body,
        mesh=mesh,
        out_type=[
            jax.ShapeDtypeStruct((TP1, B, U), jnp.float32),
            jax.ShapeDtypeStruct((B * TP1,), jnp.float32),
        ],
        scratch_types=[
            pltpu.VMEM((EPT * NNZ_SENT,), jnp.int32),
            pltpu.VMEM((128, D), jnp.float32),
            pltpu.VMEM((32, DU), jnp.float32),
            pltpu.VMEM((32, L), jnp.int32),
            pltpu.VMEM((32, U), jnp.float32),
            pltpu.VMEM((EPT * TP1 + L,), jnp.float32),
            pltpu.VMEM((L, 128), jnp.int32),
            pltpu.VMEM((CB, D), jnp.float32),
            pltpu.VMEM((CB, D), jnp.float32),
            pltpu.VMEM((CB, DU), jnp.float32),
            pltpu.VMEM((CB, U), jnp.float32),
            pltpu.VMEM((CB, L), jnp.int32),
            pltpu.VMEM((CB, U), jnp.float32),
            pltpu.VMEM_SHARED((B, U), jnp.float32),
            pltpu.SemaphoreType.DMA,
            pltpu.SemaphoreType.DMA,
            pltpu.SemaphoreType.DMA,
        ],
    )(seqi3, seqd_t, sent_idx, sent_dense, lens_b, wseq, wsent)


def kernel(seq_sparse_idx, seq_dense, sent_sparse_idx, sent_dense,
           sequence_feature_lengths, W_seq, W_sent):
    seqi3 = (seq_sparse_idx.astype(jnp.int32)
             .transpose(1, 0, 2)
             .reshape(T, B // 128, 128, NNZ_SEQ)
             .transpose(0, 1, 3, 2)
             .reshape(T, 2 * (B // 128), 128))
    seqd_t = seq_dense.transpose(1, 0, 2)
    sent_idx = sent_sparse_idx.reshape(B * NNZ_SENT).astype(jnp.int32)
    sent_dense2 = sent_dense.reshape(B, DU)
    lens = sequence_feature_lengths.astype(jnp.int32)
    lens_b = jnp.broadcast_to(lens[:, None], (B, L))
    comb_t, mask_flat = _run(seqi3, seqd_t, sent_idx, sent_dense2,
                             lens_b, W_seq, W_sent)
    return comb_t.transpose(1, 0, 2), mask_flat.reshape(B, TP1, 1)


# speedup vs baseline: 2.3122x; 2.3122x over previous
import jax
import jax.numpy as jnp
from jax import lax
from jax.experimental import pallas as pl
from jax.experimental.pallas import tpu as pltpu
from jax.experimental.pallas import tpu_sc as plsc

B, T, V, D, DU = 1024, 50, 100000, 128, 256
U = D + DU
TP1 = T + 1
NNZ_SEQ = 2
NNZ_SENT = 4
NW = 32
L = 16
CB = 32


def _sc_body(seqi3, seqd_t, sent_idx, sent_dense, lens, lens_b,
             wseq, wsent, comb2, masko,
             sidx_st, gsh, dsh, lensh, lens_c, srow_st, sidxbuf, mst,
             idx0, idx1, g00, g01, g10, g11, dc0, dc1, o0, o1, zbuf,
             s1, si0, si1, sg00, sg01, sg10, sg11, sd0, sd1, so0, so1,
             szb, ssc):
    cid = lax.axis_index("c")
    sid = lax.axis_index("s")
    wid = sid * 2 + cid
    bb = pl.multiple_of(wid * CB, CB)
    cq = wid // 4
    r0 = (wid - cq * 4) * CB

    idxb = (idx0, idx1)
    gz = ((g00, g01), (g10, g11))
    dbuf = (dc0, dc1)
    obuf = (o0, o1)
    sib = (si0, si1)
    sgz = ((sg00, sg01), (sg10, sg11))
    sdb = (sd0, sd1)
    sob = (so0, so1)

    iota = jnp.arange(L, dtype=jnp.int32)
    zero_f = jnp.zeros((L,), jnp.float32)
    one_f = jnp.float32(1.0)
    zero_s = jnp.float32(0.0)

    pltpu.sync_copy(sent_idx.at[pl.ds(bb * NNZ_SENT, CB * NNZ_SENT)],
                    sidx_st)
    pltpu.async_copy(wsent.at[sidx_st], gsh, s1).wait()
    pltpu.sync_copy(sent_dense.at[pl.ds(bb, CB)], dsh)
    pltpu.sync_copy(lens_b.at[pl.ds(bb, CB)], lensh)
    pltpu.sync_copy(lens.at[pl.ds(bb, CB)], lens_c)

    def ph1_body(j2, carry):
        for cc in range(D // L):
            srow_st[j2, pl.ds(cc * L, L)] = (
                (gsh[NNZ_SENT * j2, pl.ds(cc * L, L)]
                 + gsh[NNZ_SENT * j2 + 1, pl.ds(cc * L, L)])
                + (gsh[NNZ_SENT * j2 + 2, pl.ds(cc * L, L)]
                   + gsh[NNZ_SENT * j2 + 3, pl.ds(cc * L, L)]))
        for cc in range(DU // L):
            srow_st[j2, pl.ds(D + cc * L, L)] = dsh[j2, pl.ds(cc * L, L)]
        len_v = lensh[j2, pl.ds(0, L)]
        lp1 = jnp.full((L,), 1, jnp.int32) + len_v
        for kk in range(4):
            tvec = kk * L + iota
            m = jnp.where(tvec < lp1, one_f, zero_s)
            mst[pl.ds(j2 * TP1 + kk * L, L)] = m
        for cc in range(U // L):
            zbuf[j2, pl.ds(cc * L, L)] = zero_f
        return carry

    lax.fori_loop(0, CB, ph1_body, 0)

    for k2 in range(CB // L):
        lv = lens_c[pl.ds(k2 * L, L)]
        sidxbuf[pl.ds(k2 * L, L)] = lv * B + (bb + k2 * L + iota)

    pltpu.async_copy(zbuf, comb2.at[pl.ds(pl.multiple_of(T * B + bb, 8),
                                          CB)], szb)

    def idx_cp(k, t):
        return pltpu.make_async_copy(seqi3.at[t], idxb[k], sib[k])

    def g_cp(z, k):
        return pltpu.make_async_copy(
            wseq.at[idxb[k].at[2 * cq + z, pl.ds(r0, CB)]], gz[z][k],
            sgz[z][k])

    def d_cp(k, t):
        return pltpu.make_async_copy(
            seqd_t.at[t, pl.ds(bb, CB)], dbuf[k], sdb[k])

    def o_cp(k, t):
        return pltpu.make_async_copy(
            obuf[k], comb2.at[pl.ds(pl.multiple_of(t * B + bb, 8), CB)],
            sob[k])

    idx_cp(0, 0).start()
    idx_cp(1, 1).start()
    idx_cp(0, 0).wait()
    g_cp(0, 0).start()
    g_cp(1, 0).start()
    d_cp(0, 0).start()

    def pair_body(p, carry):
        for k in range(2):
            t = 2 * p + k
            ko = 1 - k
            g_cp(0, k).wait()
            g_cp(1, k).wait()
            d_cp(k, t).wait()

            @pl.when(t + 2 < T)
            def _():
                idx_cp(k, t + 2).start()

            @pl.when(t + 1 < T)
            def _():
                idx_cp(ko, t + 1).wait()
                g_cp(0, ko).start()
                g_cp(1, ko).start()
                d_cp(ko, t + 1).start()

            @pl.when(t >= 2)
            def _():
                o_cp(k, t - 2).wait()

            tv = jnp.full((L,), t, jnp.int32)
            g0k, g1k, dk, ok = gz[0][k], gz[1][k], dbuf[k], obuf[k]

            @plsc.parallel_loop(0, CB, unroll=2)
            def row_body(bl):
                len_v = lensh[bl, pl.ds(0, L)]
                m_lt = tv < len_v
                for cc in range(D // L):
                    v = g0k[bl, pl.ds(cc * L, L)] + g1k[bl, pl.ds(cc * L, L)]
                    ok[bl, pl.ds(cc * L, L)] = jnp.where(m_lt, v, zero_f)
                for cc in range(DU // L):
                    ok[bl, pl.ds(D + cc * L, L)] = jnp.where(
                        m_lt, dk[bl, pl.ds(cc * L, L)], zero_f)

            o_cp(k, t).start()
        return carry

    lax.fori_loop(0, T // 2, pair_body, 0)

    o_cp(0, T - 2).wait()
    o_cp(1, T - 1).wait()
    pltpu.make_async_copy(
        zbuf, comb2.at[pl.ds(pl.multiple_of(T * B + bb, 8), CB)],
        szb).wait()
    pltpu.async_copy(srow_st, comb2.at[sidxbuf], ssc).wait()
    pltpu.sync_copy(mst.at[pl.ds(0, CB * TP1)],
                    masko.at[pl.ds(bb * TP1, CB * TP1)])


@jax.jit
def _run(seqi3, seqd_t, sent_idx, sent_dense, lens, lens_b, wseq, wsent):
    mesh = plsc.VectorSubcoreMesh(core_axis_name="c", subcore_axis_name="s")
    return pl.kernel(
        _sc_body,
        mesh=mesh,
        out_type=[
            jax.ShapeDtypeStruct((TP1 * B, U), jnp.float32),
            jax.ShapeDtypeStruct((B * TP1,), jnp.float32),
        ],
        scratch_types=[
            pltpu.VMEM((CB * NNZ_SENT,), jnp.int32),
            pltpu.VMEM((CB * NNZ_SENT, D), jnp.float32),
            pltpu.VMEM((CB, DU), jnp.float32),
            pltpu.VMEM((CB, L), jnp.int32),
            pltpu.VMEM((CB,), jnp.int32),
            pltpu.VMEM((CB, U), jnp.float32),
            pltpu.VMEM((CB,), jnp.int32),
            pltpu.VMEM((CB * TP1 + L,), jnp.float32),
            pltpu.VMEM((L, 128), jnp.int32),
            pltpu.VMEM((L, 128), jnp.int32),
            pltpu.VMEM((CB, D), jnp.float32),
            pltpu.VMEM((CB, D), jnp.float32),
            pltpu.VMEM((CB, D), jnp.float32),
            pltpu.VMEM((CB, D), jnp.float32),
            pltpu.VMEM((CB, DU), jnp.float32),
            pltpu.VMEM((CB, DU), jnp.float32),
            pltpu.VMEM((CB, U), jnp.float32),
            pltpu.VMEM((CB, U), jnp.float32),
            pltpu.VMEM((CB, U), jnp.float32),
        ] + [pltpu.SemaphoreType.DMA] * 13,
    )(seqi3, seqd_t, sent_idx, sent_dense, lens, lens_b, wseq, wsent)


def kernel(seq_sparse_idx, seq_dense, sent_sparse_idx, sent_dense,
           sequence_feature_lengths, W_seq, W_sent):
    seqi3 = (seq_sparse_idx.astype(jnp.int32)
             .transpose(1, 0, 2)
             .reshape(T, B // 128, 128, NNZ_SEQ)
             .transpose(0, 1, 3, 2)
             .reshape(T, NNZ_SEQ * (B // 128), 128))
    seqd_t = seq_dense.transpose(1, 0, 2)
    sent_idx = sent_sparse_idx.reshape(B * NNZ_SENT).astype(jnp.int32)
    sent_dense2 = sent_dense.reshape(B, DU)
    lens = sequence_feature_lengths.astype(jnp.int32)
    lens_b = jnp.broadcast_to(lens[:, None], (B, L))
    comb2, mask_flat = _run(seqi3, seqd_t, sent_idx, sent_dense2,
                            lens, lens_b, W_seq, W_sent)
    comb = comb2.reshape(TP1, B, U).transpose(1, 0, 2)
    return comb, mask_flat.reshape(B, TP1, 1)


# speedup vs baseline: 2.6169x; 1.1318x over previous
import jax
import jax.numpy as jnp
from jax import lax
from jax.experimental import pallas as pl
from jax.experimental.pallas import tpu as pltpu
from jax.experimental.pallas import tpu_sc as plsc

B, T, V, D, DU = 1024, 50, 100000, 128, 256
U = D + DU
TP1 = T + 1
NNZ_SEQ = 2
NNZ_SENT = 4
NW = 32
L = 16
CB = 32


def _sc_body(seqi3, seqd_t, sent_idx, sent_dense, lens, lens_b,
             wseq, wsent, comb2, masko,
             sidx_st, gsh, dsh, lensh, lens_c, srow_st, sidxbuf, mst,
             idx0, idx1, g00, g01, g10, g11, dc0, dc1, o0, o1, zbuf,
             s1, si0, si1, sg00, sg01, sg10, sg11, sd0, sd1, so0, so1,
             szb, ssc):
    cid = lax.axis_index("c")
    sid = lax.axis_index("s")
    wid = sid * 2 + cid
    bb = pl.multiple_of(wid * CB, CB)
    cq = wid // 4
    r0 = (wid - cq * 4) * CB

    idxb = (idx0, idx1)
    gz = ((g00, g01), (g10, g11))
    dbuf = (dc0, dc1)
    obuf = (o0, o1)
    sib = (si0, si1)
    sgz = ((sg00, sg01), (sg10, sg11))
    sdb = (sd0, sd1)
    sob = (so0, so1)

    iota = jnp.arange(L, dtype=jnp.int32)
    zero_f = jnp.zeros((L,), jnp.float32)
    one_f = jnp.float32(1.0)
    zero_s = jnp.float32(0.0)

    pltpu.sync_copy(sent_idx.at[pl.ds(bb * NNZ_SENT, CB * NNZ_SENT)],
                    sidx_st)
    pltpu.async_copy(wsent.at[sidx_st], gsh, s1).wait()
    pltpu.sync_copy(sent_dense.at[pl.ds(bb, CB)], dsh)
    pltpu.sync_copy(lens_b.at[pl.ds(bb, CB)], lensh)
    pltpu.sync_copy(lens.at[pl.ds(bb, CB)], lens_c)

    def ph1_body(j2, carry):
        for cc in range(D // L):
            srow_st[j2, pl.ds(cc * L, L)] = (
                (gsh[NNZ_SENT * j2, pl.ds(cc * L, L)]
                 + gsh[NNZ_SENT * j2 + 1, pl.ds(cc * L, L)])
                + (gsh[NNZ_SENT * j2 + 2, pl.ds(cc * L, L)]
                   + gsh[NNZ_SENT * j2 + 3, pl.ds(cc * L, L)]))
        for cc in range(DU // L):
            srow_st[j2, pl.ds(D + cc * L, L)] = dsh[j2, pl.ds(cc * L, L)]
        len_v = lensh[j2, pl.ds(0, L)]
        lp1 = jnp.full((L,), 1, jnp.int32) + len_v
        for kk in range(4):
            tvec = kk * L + iota
            m = jnp.where(tvec < lp1, one_f, zero_s)
            mst[pl.ds(j2 * TP1 + kk * L, L)] = m
        for cc in range(U // L):
            zbuf[j2, pl.ds(cc * L, L)] = zero_f
        return carry

    lax.fori_loop(0, CB, ph1_body, 0)

    for k2 in range(CB // L):
        lv = lens_c[pl.ds(k2 * L, L)]
        sidxbuf[pl.ds(k2 * L, L)] = lv * B + (bb + k2 * L + iota)

    pltpu.async_copy(zbuf, comb2.at[pl.ds(pl.multiple_of(T * B + bb, 8),
                                          CB)], szb)

    def idx_cp(k, t):
        return pltpu.make_async_copy(seqi3.at[t, pl.ds(cq, 1)], idxb[k],
                                     sib[k])

    def g_cp(z, k):
        return pltpu.make_async_copy(
            wseq.at[idxb[k].at[0, z, pl.ds(r0, CB)]], gz[z][k], sgz[z][k])

    def d_cp(k, t):
        return pltpu.make_async_copy(
            seqd_t.at[t, pl.ds(bb, CB)], dbuf[k], sdb[k])

    def o_cp(k, t):
        return pltpu.make_async_copy(
            obuf[k], comb2.at[pl.ds(pl.multiple_of(t * B + bb, 8), CB)],
            sob[k])

    idx_cp(0, 0).start()
    idx_cp(1, 1).start()
    idx_cp(0, 0).wait()
    g_cp(0, 0).start()
    g_cp(1, 0).start()
    d_cp(0, 0).start()

    def pair_body(p, carry):
        for k in range(2):
            t = 2 * p + k
            ko = 1 - k
            g_cp(0, k).wait()
            g_cp(1, k).wait()
            d_cp(k, t).wait()

            @pl.when(t + 2 < T)
            def _():
                idx_cp(k, t + 2).start()

            @pl.when(t + 1 < T)
            def _():
                idx_cp(ko, t + 1).wait()
                g_cp(0, ko).start()
                g_cp(1, ko).start()
                d_cp(ko, t + 1).start()

            @pl.when(t >= 2)
            def _():
                o_cp(k, t - 2).wait()

            tv = jnp.full((L,), t, jnp.int32)
            g0k, g1k, dk, ok = gz[0][k], gz[1][k], dbuf[k], obuf[k]

            @plsc.parallel_loop(0, CB, unroll=2)
            def row_body(bl):
                len_v = lensh[bl, pl.ds(0, L)]
                m_lt = tv < len_v
                for cc in range(D // L):
                    v = g0k[bl, pl.ds(cc * L, L)] + g1k[bl, pl.ds(cc * L, L)]
                    ok[bl, pl.ds(cc * L, L)] = jnp.where(m_lt, v, zero_f)
                for cc in range(DU // L):
                    ok[bl, pl.ds(D + cc * L, L)] = jnp.where(
                        m_lt, dk[bl, pl.ds(cc * L, L)], zero_f)

            o_cp(k, t).start()
        return carry

    lax.fori_loop(0, T // 2, pair_body, 0)

    o_cp(0, T - 2).wait()
    o_cp(1, T - 1).wait()
    pltpu.make_async_copy(
        zbuf, comb2.at[pl.ds(pl.multiple_of(T * B + bb, 8), CB)],
        szb).wait()
    pltpu.async_copy(srow_st, comb2.at[sidxbuf], ssc).wait()
    pltpu.sync_copy(mst.at[pl.ds(0, CB * TP1)],
                    masko.at[pl.ds(bb * TP1, CB * TP1)])


@jax.jit
def _run(seqi3, seqd_t, sent_idx, sent_dense, lens, lens_b, wseq, wsent):
    mesh = plsc.VectorSubcoreMesh(core_axis_name="c", subcore_axis_name="s")
    return pl.kernel(
        _sc_body,
        mesh=mesh,
        out_type=[
            jax.ShapeDtypeStruct((TP1 * B, U), jnp.float32),
            jax.ShapeDtypeStruct((B * TP1,), jnp.float32),
        ],
        scratch_types=[
            pltpu.VMEM((CB * NNZ_SENT,), jnp.int32),
            pltpu.VMEM((CB * NNZ_SENT, D), jnp.float32),
            pltpu.VMEM((CB, DU), jnp.float32),
            pltpu.VMEM((CB, L), jnp.int32),
            pltpu.VMEM((CB,), jnp.int32),
            pltpu.VMEM((CB, U), jnp.float32),
            pltpu.VMEM((CB,), jnp.int32),
            pltpu.VMEM((CB * TP1 + L,), jnp.float32),
            pltpu.VMEM((1, NNZ_SEQ, 128), jnp.int32),
            pltpu.VMEM((1, NNZ_SEQ, 128), jnp.int32),
            pltpu.VMEM((CB, D), jnp.float32),
            pltpu.VMEM((CB, D), jnp.float32),
            pltpu.VMEM((CB, D), jnp.float32),
            pltpu.VMEM((CB, D), jnp.float32),
            pltpu.VMEM((CB, DU), jnp.float32),
            pltpu.VMEM((CB, DU), jnp.float32),
            pltpu.VMEM((CB, U), jnp.float32),
            pltpu.VMEM((CB, U), jnp.float32),
            pltpu.VMEM((CB, U), jnp.float32),
        ] + [pltpu.SemaphoreType.DMA] * 13,
    )(seqi3, seqd_t, sent_idx, sent_dense, lens, lens_b, wseq, wsent)


def kernel(seq_sparse_idx, seq_dense, sent_sparse_idx, sent_dense,
           sequence_feature_lengths, W_seq, W_sent):
    seqi3 = (seq_sparse_idx.astype(jnp.int32)
             .transpose(1, 0, 2)
             .reshape(T, B // 128, 128, NNZ_SEQ)
             .transpose(0, 1, 3, 2))
    seqd_t = seq_dense.transpose(1, 0, 2)
    sent_idx = sent_sparse_idx.reshape(B * NNZ_SENT).astype(jnp.int32)
    sent_dense2 = sent_dense.reshape(B, DU)
    lens = sequence_feature_lengths.astype(jnp.int32)
    lens_b = jnp.broadcast_to(lens[:, None], (B, L))
    comb2, mask_flat = _run(seqi3, seqd_t, sent_idx, sent_dense2,
                            lens, lens_b, W_seq, W_sent)
    comb = comb2.reshape(TP1, B, U).transpose(1, 0, 2)
    return comb, mask_flat.reshape(B, TP1, 1)


# speedup vs baseline: 2.6747x; 1.0221x over previous
import jax
import jax.numpy as jnp
from jax import lax
from jax.experimental import pallas as pl
from jax.experimental.pallas import tpu as pltpu
from jax.experimental.pallas import tpu_sc as plsc

B, T, V, D, DU = 1024, 50, 100000, 128, 256
U = D + DU
TP1 = T + 1
NNZ_SEQ = 2
NNZ_SENT = 4
NW = 32
L = 16
CB = 32


def _sc_body(seqi3, seqd_t, sent_idx, sent_dense, lens, lens_b,
             wseq, wsent, comb2, masko,
             sidx_st, gsh, dsh, lensh, lens_c, srow_st, sidxbuf, mst,
             idx0, idx1, g00, g01, g10, g11, dc0, dc1, o0, o1, zbuf,
             s1, si0, si1, sg00, sg01, sg10, sg11, sd0, sd1, so0, so1,
             szb, ssc):
    cid = lax.axis_index("c")
    sid = lax.axis_index("s")
    wid = sid * 2 + cid
    bb = pl.multiple_of(wid * CB, CB)
    cq = wid // 4
    r0 = (wid - cq * 4) * CB

    idxb = (idx0, idx1)
    gz = ((g00, g01), (g10, g11))
    dbuf = (dc0, dc1)
    obuf = (o0, o1)
    sib = (si0, si1)
    sgz = ((sg00, sg01), (sg10, sg11))
    sdb = (sd0, sd1)
    sob = (so0, so1)

    iota = jnp.arange(L, dtype=jnp.int32)
    zero_f = jnp.zeros((L,), jnp.float32)
    one_f = jnp.float32(1.0)
    zero_s = jnp.float32(0.0)

    def idx_cp(k, t):
        return pltpu.make_async_copy(seqi3.at[t, pl.ds(cq, 1)], idxb[k],
                                     sib[k])

    def g_cp(z, k):
        return pltpu.make_async_copy(
            wseq.at[idxb[k].at[0, z, pl.ds(r0, CB)]], gz[z][k], sgz[z][k])

    def d_cp(k, t):
        return pltpu.make_async_copy(
            seqd_t.at[t, pl.ds(bb, CB)], dbuf[k], sdb[k])

    def o_cp(k, t):
        return pltpu.make_async_copy(
            obuf[k], comb2.at[pl.ds(pl.multiple_of(t * B + bb, 8), CB)],
            sob[k])

    idx_cp(0, 0).start()
    idx_cp(1, 1).start()
    idx_cp(0, 0).wait()
    g_cp(0, 0).start()
    g_cp(1, 0).start()
    d_cp(0, 0).start()

    pltpu.sync_copy(sent_idx.at[pl.ds(bb * NNZ_SENT, CB * NNZ_SENT)],
                    sidx_st)
    pltpu.async_copy(wsent.at[sidx_st], gsh, s1).wait()
    pltpu.sync_copy(sent_dense.at[pl.ds(bb, CB)], dsh)
    pltpu.sync_copy(lens_b.at[pl.ds(bb, CB)], lensh)
    pltpu.sync_copy(lens.at[pl.ds(bb, CB)], lens_c)

    def ph1_body(j2, carry):
        for cc in range(D // L):
            srow_st[j2, pl.ds(cc * L, L)] = (
                (gsh[NNZ_SENT * j2, pl.ds(cc * L, L)]
                 + gsh[NNZ_SENT * j2 + 1, pl.ds(cc * L, L)])
                + (gsh[NNZ_SENT * j2 + 2, pl.ds(cc * L, L)]
                   + gsh[NNZ_SENT * j2 + 3, pl.ds(cc * L, L)]))
        for cc in range(DU // L):
            srow_st[j2, pl.ds(D + cc * L, L)] = dsh[j2, 0, pl.ds(cc * L, L)]
        len_v = lensh[j2, pl.ds(0, L)]
        lp1 = jnp.full((L,), 1, jnp.int32) + len_v
        for kk in range(4):
            tvec = kk * L + iota
            m = jnp.where(tvec < lp1, one_f, zero_s)
            mst[pl.ds(j2 * TP1 + kk * L, L)] = m
        for cc in range(U // L):
            zbuf[j2, pl.ds(cc * L, L)] = zero_f
        return carry

    lax.fori_loop(0, CB, ph1_body, 0)

    for k2 in range(CB // L):
        lv = lens_c[pl.ds(k2 * L, L)]
        sidxbuf[pl.ds(k2 * L, L)] = lv * B + (bb + k2 * L + iota)

    pltpu.async_copy(zbuf, comb2.at[pl.ds(pl.multiple_of(T * B + bb, 8),
                                          CB)], szb)


    def pair_body(p, carry):
        for k in range(2):
            t = 2 * p + k
            ko = 1 - k
            g_cp(0, k).wait()
            g_cp(1, k).wait()
            d_cp(k, t).wait()

            @pl.when(t + 2 < T)
            def _():
                idx_cp(k, t + 2).start()

            @pl.when(t + 1 < T)
            def _():
                idx_cp(ko, t + 1).wait()
                g_cp(0, ko).start()
                g_cp(1, ko).start()
                d_cp(ko, t + 1).start()

            @pl.when(t >= 2)
            def _():
                o_cp(k, t - 2).wait()

            tv = jnp.full((L,), t, jnp.int32)
            g0k, g1k, dk, ok = gz[0][k], gz[1][k], dbuf[k], obuf[k]

            @plsc.parallel_loop(0, CB, unroll=2)
            def row_body(bl):
                len_v = lensh[bl, pl.ds(0, L)]
                m_lt = tv < len_v
                for cc in range(D // L):
                    v = g0k[bl, pl.ds(cc * L, L)] + g1k[bl, pl.ds(cc * L, L)]
                    ok[bl, pl.ds(cc * L, L)] = jnp.where(m_lt, v, zero_f)
                for cc in range(DU // L):
                    ok[bl, pl.ds(D + cc * L, L)] = jnp.where(
                        m_lt, dk[bl, pl.ds(cc * L, L)], zero_f)

            o_cp(k, t).start()
        return carry

    lax.fori_loop(0, T // 2, pair_body, 0)

    o_cp(0, T - 2).wait()
    o_cp(1, T - 1).wait()
    pltpu.make_async_copy(
        zbuf, comb2.at[pl.ds(pl.multiple_of(T * B + bb, 8), CB)],
        szb).wait()
    pltpu.async_copy(srow_st, comb2.at[sidxbuf], ssc).wait()
    pltpu.sync_copy(mst.at[pl.ds(0, CB * TP1)],
                    masko.at[pl.ds(bb * TP1, CB * TP1)])


@jax.jit
def _run(seqi3, seqd_t, sent_idx, sent_dense, lens, lens_b, wseq, wsent):
    mesh = plsc.VectorSubcoreMesh(core_axis_name="c", subcore_axis_name="s")
    return pl.kernel(
        _sc_body,
        mesh=mesh,
        out_type=[
            jax.ShapeDtypeStruct((TP1 * B, U), jnp.float32),
            jax.ShapeDtypeStruct((B * TP1,), jnp.float32),
        ],
        scratch_types=[
            pltpu.VMEM((CB * NNZ_SENT,), jnp.int32),
            pltpu.VMEM((CB * NNZ_SENT, D), jnp.float32),
            pltpu.VMEM((CB, 1, DU), jnp.float32),
            pltpu.VMEM((CB, L), jnp.int32),
            pltpu.VMEM((CB,), jnp.int32),
            pltpu.VMEM((CB, U), jnp.float32),
            pltpu.VMEM((CB,), jnp.int32),
            pltpu.VMEM((CB * TP1 + L,), jnp.float32),
            pltpu.VMEM((1, NNZ_SEQ, 128), jnp.int32),
            pltpu.VMEM((1, NNZ_SEQ, 128), jnp.int32),
            pltpu.VMEM((CB, D), jnp.float32),
            pltpu.VMEM((CB, D), jnp.float32),
            pltpu.VMEM((CB, D), jnp.float32),
            pltpu.VMEM((CB, D), jnp.float32),
            pltpu.VMEM((CB, DU), jnp.float32),
            pltpu.VMEM((CB, DU), jnp.float32),
            pltpu.VMEM((CB, U), jnp.float32),
            pltpu.VMEM((CB, U), jnp.float32),
            pltpu.VMEM((CB, U), jnp.float32),
        ] + [pltpu.SemaphoreType.DMA] * 13,
    )(seqi3, seqd_t, sent_idx, sent_dense, lens, lens_b, wseq, wsent)


def kernel(seq_sparse_idx, seq_dense, sent_sparse_idx, sent_dense,
           sequence_feature_lengths, W_seq, W_sent):
    seqi3 = (seq_sparse_idx.astype(jnp.int32)
             .transpose(1, 0, 2)
             .reshape(T, B // 128, 128, NNZ_SEQ)
             .transpose(0, 1, 3, 2))
    seqd_t = seq_dense.transpose(1, 0, 2)
    sent_idx = sent_sparse_idx.reshape(B * NNZ_SENT).astype(jnp.int32)
    lens = sequence_feature_lengths.astype(jnp.int32)
    lens_b = jnp.broadcast_to(lens[:, None], (B, L))
    comb2, mask_flat = _run(seqi3, seqd_t, sent_idx, sent_dense,
                            lens, lens_b, W_seq, W_sent)
    comb = comb2.reshape(TP1, B, U).transpose(1, 0, 2)
    return comb, mask_flat.reshape(B, TP1, 1)
